# baseline (speedup 1.0000x reference)
import jax
import jax.numpy as jnp
from jax.experimental import pallas as pl
from jax.experimental.pallas import tpu as pltpu

_B = 128
_LIN = 1152
_ACT1 = 896
_NP2 = 600
_NPX = 592


def _sigmoid(z):
    e = jnp.exp(-jnp.abs(z))
    inv = 1.0 / (1.0 + e)
    return jnp.where(z >= 0.0, inv, e * inv)


def _net_kernel(xt_ref, w1_ref, b1_ref, w2_ref, b2_ref, wl_ref, bl_ref,
                o_ref, patch1, z1, p2in, patchx, z2c, h):
    for ky in range(5):
        for kx in range(5):
            patch1[ky * 5 + kx] = xt_ref[pl.ds(ky * 32 + kx, _ACT1), :]
    z1[...] = (jnp.dot(w1_ref[...], patch1[...].reshape(25, _ACT1 * _B),
                       preferred_element_type=jnp.float32)
               + b1_ref[...]).reshape(16, _ACT1, _B)

    a = z1[:, pl.ds(0, 430, 2), :]
    b = z1[:, pl.ds(1, 430, 2), :]
    c = z1[:, pl.ds(32, 430, 2), :]
    d = z1[:, pl.ds(33, 430, 2), :]
    s1 = _sigmoid(jnp.maximum(jnp.maximum(a, b), jnp.maximum(c, d)))
    li = jax.lax.broadcasted_iota(jnp.int32, (16, 430, _B), 1)
    s1 = jnp.where((li & 31) < 14, s1, 0.0)
    p2in[:, pl.ds(0, 72), :] = jnp.zeros((16, 72, _B), jnp.float32)
    p2in[:, pl.ds(66, 430), :] = s1
    p2in[:, pl.ds(496, _NP2 - 496), :] = jnp.zeros((16, _NP2 - 496, _B),
                                                   jnp.float32)

    for kx in range(5):
        patchx[pl.ds(kx * 16, 16), :, :] = (
            p2in[:, pl.ds(kx, _NPX), :].astype(jnp.bfloat16))

    h[...] = jnp.zeros((32, 256, _B), jnp.float32)
    for k in range(7):
        zk = b2_ref[...]
        for ky in range(5):
            rhs = patchx[:, pl.ds(64 * k + 32 * ky, 64), :].reshape(80, 64 * _B)
            zk = zk + jnp.dot(w2_ref[ky], rhs,
                              preferred_element_type=jnp.float32)
        z2c[...] = zk.reshape(32, 64, _B)
        a2 = z2c[:, pl.ds(0, 16, 2), :]
        b2 = z2c[:, pl.ds(1, 16, 2), :]
        c2 = z2c[:, pl.ds(32, 16, 2), :]
        d2 = z2c[:, pl.ds(33, 16, 2), :]
        h[:, pl.ds(32 * k, 16), :] = _sigmoid(
            jnp.maximum(jnp.maximum(a2, b2), jnp.maximum(c2, d2)))

    o_ref[...] = (jnp.dot(wl_ref[...], h[...].reshape(32 * 256, _B),
                          preferred_element_type=jnp.float32)
                  + bl_ref[...])


def kernel(x, w1, b1, w2, b2, wlin, blin, sel1, sel2,
           raw_w1, raw_b1, raw_w2, raw_b2, raw_wout, raw_bout):
    n = x.shape[0]
    n_pad = _B * ((n + _B - 1) // _B)
    x3 = x.reshape(n, 28, 28)
    if n_pad != n:
        x3 = jnp.pad(x3, ((0, n_pad - n), (0, 0), (0, 0)))
    xp = jnp.pad(x3, ((0, 0), (2, 6), (2, 2))).reshape(n_pad, _LIN)
    xt = xp.T

    w2r = w2.reshape(32, 5, 80).transpose(1, 0, 2).astype(jnp.bfloat16)
    wl = jnp.zeros((10, 32, 8, 32), jnp.float32)
    wl = wl.at[:, :, :7, :7].set(raw_wout.reshape(10, 32, 7, 7))
    wl2 = wl.reshape(10, 32 * 256)

    out = pl.pallas_call(
        _net_kernel,
        out_shape=jax.ShapeDtypeStruct((10, n_pad), jnp.float32),
        grid=(n_pad // _B,),
        in_specs=[
            pl.BlockSpec((_LIN, _B), lambda i: (0, i)),
            pl.BlockSpec((16, 25), lambda i: (0, 0)),
            pl.BlockSpec((16, 1), lambda i: (0, 0)),
            pl.BlockSpec((5, 32, 80), lambda i: (0, 0, 0)),
            pl.BlockSpec((32, 1), lambda i: (0, 0)),
            pl.BlockSpec((10, 32 * 256), lambda i: (0, 0)),
            pl.BlockSpec((10, 1), lambda i: (0, 0)),
        ],
        out_specs=pl.BlockSpec((10, _B), lambda i: (0, i)),
        scratch_shapes=[
            pltpu.VMEM((25, _ACT1, _B), jnp.float32),
            pltpu.VMEM((16, _ACT1, _B), jnp.float32),
            pltpu.VMEM((16, _NP2, _B), jnp.float32),
            pltpu.VMEM((80, _NPX, _B), jnp.bfloat16),
            pltpu.VMEM((32, 64, _B), jnp.float32),
            pltpu.VMEM((32, 256, _B), jnp.float32),
        ],
        compiler_params=pltpu.CompilerParams(
            dimension_semantics=("parallel",),
            vmem_limit_bytes=64 * 1024 * 1024,
        ),
    )(xt, w1, b1, w2r, b2, wl2, blin.T)
    return out.T[:n]


# speedup vs baseline: 1.8368x; 1.8368x over previous
import jax
import jax.numpy as jnp
from jax.experimental import pallas as pl
from jax.experimental.pallas import tpu as pltpu

_B = 128
_LIN = 1152
_ACT1 = 896
_NP2 = 648
_NPX = 640


def _sigmoid(z):
    e = jnp.exp(-jnp.abs(z))
    inv = 1.0 / (1.0 + e)
    return jnp.where(z >= 0.0, inv, e * inv)


def _net_kernel(xt_ref, w1_ref, b1_ref, w2_ref, b2_ref, wl_ref, bl_ref,
                o_ref, patch1, z1, p2in, patchx, z2c, h):
    for ky in range(5):
        for kx in range(5):
            patch1[ky * 5 + kx] = xt_ref[pl.ds(ky * 32 + kx, _ACT1), :]
    z1[...] = (jnp.dot(w1_ref[...], patch1[...].reshape(25, _ACT1 * _B),
                       preferred_element_type=jnp.float32)
               + b1_ref[...]).reshape(16, _ACT1, _B)

    a = z1[:, pl.ds(0, 430, 2), :]
    b = z1[:, pl.ds(1, 430, 2), :]
    c = z1[:, pl.ds(32, 430, 2), :]
    d = z1[:, pl.ds(33, 430, 2), :]
    s1 = _sigmoid(jnp.maximum(jnp.maximum(a, b), jnp.maximum(c, d)))
    li = jax.lax.broadcasted_iota(jnp.int32, (16, 430, _B), 1)
    s1 = jnp.where((li & 31) < 14, s1, 0.0)
    p2in[:, pl.ds(0, 72), :] = jnp.zeros((16, 72, _B), jnp.float32)
    p2in[:, pl.ds(66, 430), :] = s1
    p2in[:, pl.ds(496, _NP2 - 496), :] = jnp.zeros((16, _NP2 - 496, _B),
                                                   jnp.float32)

    for kx in range(5):
        patchx[pl.ds(kx * 16, 16), :, :] = p2in[:, pl.ds(kx, _NPX), :]

    h[...] = jnp.zeros((32, 256, _B), jnp.float32)
    for k in range(4):
        zk = b2_ref[...]
        for ky in range(5):
            rhs = patchx[:, pl.ds(128 * k + 32 * ky, 128), :].reshape(
                80, 128 * _B)
            zk = zk + jnp.dot(w2_ref[ky], rhs,
                              preferred_element_type=jnp.float32)
        z2c[...] = zk.reshape(32, 128, _B)
        a2 = z2c[:, pl.ds(0, 48, 2), :]
        b2 = z2c[:, pl.ds(1, 48, 2), :]
        c2 = z2c[:, pl.ds(32, 48, 2), :]
        d2 = z2c[:, pl.ds(33, 48, 2), :]
        h[:, pl.ds(64 * k, 48), :] = _sigmoid(
            jnp.maximum(jnp.maximum(a2, b2), jnp.maximum(c2, d2)))

    o_ref[...] = (jnp.dot(wl_ref[...], h[...].reshape(32 * 256, _B),
                          preferred_element_type=jnp.float32)
                  + bl_ref[...])


def kernel(x, w1, b1, w2, b2, wlin, blin, sel1, sel2,
           raw_w1, raw_b1, raw_w2, raw_b2, raw_wout, raw_bout):
    n = x.shape[0]
    n_pad = _B * ((n + _B - 1) // _B)
    x3 = x.reshape(n, 28, 28)
    if n_pad != n:
        x3 = jnp.pad(x3, ((0, n_pad - n), (0, 0), (0, 0)))
    xp = jnp.pad(x3, ((0, 0), (2, 6), (2, 2))).reshape(n_pad, _LIN)
    xt = xp.T

    w2r = w2.reshape(32, 5, 80).transpose(1, 0, 2)
    wl = jnp.zeros((10, 32, 8, 32), jnp.float32)
    wl = wl.at[:, :, :7, :7].set(raw_wout.reshape(10, 32, 7, 7))
    wl2 = wl.reshape(10, 32 * 256)

    out = pl.pallas_call(
        _net_kernel,
        out_shape=jax.ShapeDtypeStruct((10, n_pad), jnp.float32),
        grid=(n_pad // _B,),
        in_specs=[
            pl.BlockSpec((_LIN, _B), lambda i: (0, i)),
            pl.BlockSpec((16, 25), lambda i: (0, 0)),
            pl.BlockSpec((16, 1), lambda i: (0, 0)),
            pl.BlockSpec((5, 32, 80), lambda i: (0, 0, 0)),
            pl.BlockSpec((32, 1), lambda i: (0, 0)),
            pl.BlockSpec((10, 32 * 256), lambda i: (0, 0)),
            pl.BlockSpec((10, 1), lambda i: (0, 0)),
        ],
        out_specs=pl.BlockSpec((10, _B), lambda i: (0, i)),
        scratch_shapes=[
            pltpu.VMEM((25, _ACT1, _B), jnp.float32),
            pltpu.VMEM((16, _ACT1, _B), jnp.float32),
            pltpu.VMEM((16, _NP2, _B), jnp.float32),
            pltpu.VMEM((80, _NPX, _B), jnp.float32),
            pltpu.VMEM((32, 128, _B), jnp.float32),
            pltpu.VMEM((32, 256, _B), jnp.float32),
        ],
        compiler_params=pltpu.CompilerParams(
            dimension_semantics=("parallel",),
            vmem_limit_bytes=64 * 1024 * 1024,
        ),
    )(xt, w1, b1, w2r, b2, wl2, blin.T)
    return out.T[:n]


# speedup vs baseline: 2.1035x; 1.1452x over previous
import jax
import jax.numpy as jnp
from jax.experimental import pallas as pl
from jax.experimental.pallas import tpu as pltpu

_B = 128
_LIN = 1152
_ACT1 = 896
_NP2 = 600
_NPX = 592


def _sigmoid(z):
    return 0.5 * jnp.tanh(0.5 * z) + 0.5


def _net_kernel(xt_ref, w1_ref, b1_ref, w2_ref, b2_ref, wl_ref, bl_ref,
                o_ref, patch1, z1, p2in, patchx, z2c, h):
    for ky in range(5):
        for kx in range(5):
            patch1[ky * 5 + kx] = xt_ref[pl.ds(ky * 32 + kx, _ACT1), :]
    z1[...] = (jnp.dot(w1_ref[...], patch1[...].reshape(25, _ACT1 * _B),
                       preferred_element_type=jnp.float32)
               + b1_ref[...]).reshape(16, _ACT1, _B)

    a = z1[:, pl.ds(0, 430, 2), :]
    b = z1[:, pl.ds(1, 430, 2), :]
    c = z1[:, pl.ds(32, 430, 2), :]
    d = z1[:, pl.ds(33, 430, 2), :]
    s1 = _sigmoid(jnp.maximum(jnp.maximum(a, b), jnp.maximum(c, d)))
    li = jax.lax.broadcasted_iota(jnp.int32, (16, 430, _B), 1)
    s1 = jnp.where((li & 31) < 14, s1, 0.0)
    p2in[:, pl.ds(0, 72), :] = jnp.zeros((16, 72, _B), jnp.float32)
    p2in[:, pl.ds(66, 430), :] = s1
    p2in[:, pl.ds(496, _NP2 - 496), :] = jnp.zeros((16, _NP2 - 496, _B),
                                                   jnp.float32)

    for kx in range(5):
        patchx[pl.ds(kx * 16, 16), :, :] = p2in[:, pl.ds(kx, _NPX), :]

    for k in range(7):
        h[:, pl.ds(32 * k + 16, 16), :] = jnp.zeros((32, 16, _B), jnp.float32)
    h[:, pl.ds(224, 32), :] = jnp.zeros((32, 32, _B), jnp.float32)
    for k in range(7):
        zk = b2_ref[...]
        for ky in range(5):
            rhs = patchx[:, pl.ds(64 * k + 32 * ky, 64), :].reshape(80, 64 * _B)
            zk = zk + jnp.dot(w2_ref[ky], rhs,
                              preferred_element_type=jnp.float32)
        z2c[...] = zk.reshape(32, 64, _B)
        a2 = z2c[:, pl.ds(0, 16, 2), :]
        b2 = z2c[:, pl.ds(1, 16, 2), :]
        c2 = z2c[:, pl.ds(32, 16, 2), :]
        d2 = z2c[:, pl.ds(33, 16, 2), :]
        h[:, pl.ds(32 * k, 16), :] = _sigmoid(
            jnp.maximum(jnp.maximum(a2, b2), jnp.maximum(c2, d2)))

    o_ref[...] = (jnp.dot(wl_ref[...], h[...].reshape(32 * 256, _B),
                          preferred_element_type=jnp.float32)
                  + bl_ref[...])


def kernel(x, w1, b1, w2, b2, wlin, blin, sel1, sel2,
           raw_w1, raw_b1, raw_w2, raw_b2, raw_wout, raw_bout):
    n = x.shape[0]
    n_pad = _B * ((n + _B - 1) // _B)
    x3 = x.reshape(n, 28, 28)
    if n_pad != n:
        x3 = jnp.pad(x3, ((0, n_pad - n), (0, 0), (0, 0)))
    xp = jnp.pad(x3, ((0, 0), (2, 6), (2, 2))).reshape(n_pad, _LIN)
    xt = xp.T

    w2r = w2.reshape(32, 5, 80).transpose(1, 0, 2)
    wl = jnp.zeros((10, 32, 8, 32), jnp.float32)
    wl = wl.at[:, :, :7, :7].set(raw_wout.reshape(10, 32, 7, 7))
    wl2 = wl.reshape(10, 32 * 256)

    out = pl.pallas_call(
        _net_kernel,
        out_shape=jax.ShapeDtypeStruct((10, n_pad), jnp.float32),
        grid=(n_pad // _B,),
        in_specs=[
            pl.BlockSpec((_LIN, _B), lambda i: (0, i)),
            pl.BlockSpec((16, 25), lambda i: (0, 0)),
            pl.BlockSpec((16, 1), lambda i: (0, 0)),
            pl.BlockSpec((5, 32, 80), lambda i: (0, 0, 0)),
            pl.BlockSpec((32, 1), lambda i: (0, 0)),
            pl.BlockSpec((10, 32 * 256), lambda i: (0, 0)),
            pl.BlockSpec((10, 1), lambda i: (0, 0)),
        ],
        out_specs=pl.BlockSpec((10, _B), lambda i: (0, i)),
        scratch_shapes=[
            pltpu.VMEM((25, _ACT1, _B), jnp.float32),
            pltpu.VMEM((16, _ACT1, _B), jnp.float32),
            pltpu.VMEM((16, _NP2, _B), jnp.float32),
            pltpu.VMEM((80, _NPX, _B), jnp.float32),
            pltpu.VMEM((32, 64, _B), jnp.float32),
            pltpu.VMEM((32, 256, _B), jnp.float32),
        ],
        compiler_params=pltpu.CompilerParams(
            dimension_semantics=("parallel",),
            vmem_limit_bytes=64 * 1024 * 1024,
        ),
    )(xt, w1, b1, w2r, b2, wl2, blin.T)
    return out.T[:n]
